# baseline (speedup 1.0000x reference)
import functools

import jax
import jax.numpy as jnp
from jax import lax
from jax.experimental import pallas as pl
from jax.experimental.pallas import tpu as pltpu
from jax.experimental.pallas import tpu_sc as plsc

B = 16384
K = 20
D = 64
LANES = 16
NC = 2
NS = 16
NW = NC * NS
BPW = B // NW
G = BPW // LANES


def _dot_rows(crows, xrows, sbuf, ssum):
    iota = lax.iota(jnp.int32, LANES)
    rots = [r * LANES + lax.bitwise_and(iota + r, LANES - 1)
            for r in range(LANES)]
    diags = [iota * LANES + lax.bitwise_and(iota + c, LANES - 1)
             for c in range(LANES)]

    def body(g, carry):
        rb = g * LANES
        for r in range(LANES):
            acc = None
            for q in range(D // (2 * LANES)):
                c32 = crows[rb + r, pl.ds(q * 2 * LANES, 2 * LANES)]
                x32 = xrows[rb + r, pl.ds(q * 2 * LANES, 2 * LANES)]
                ca, cb = plsc.unpack(c32, format=plsc.PackFormat.INTERLEAVED)
                xa, xb = plsc.unpack(x32, format=plsc.PackFormat.INTERLEAVED)
                term = ca * xa + cb * xb
                acc = term if acc is None else acc + term
            plsc.store_scatter(ssum, [rots[r]], acc)
        tot = jnp.zeros((LANES,), jnp.float32)
        for c in range(LANES):
            tot = tot + plsc.load_gather(ssum, [diags[c]])
        sbuf[pl.ds(rb, LANES)] = tot
        return carry

    lax.fori_loop(0, G, body, 0)


IDX_N = (K + 2) * BPW


def _sc_scores_body(center_h, context_h, negT_h, wc_h, wx_h,
                    pos_out, neg_out,
                    idxs, crows, xrows_a, xrows_b, sbuf, ssum,
                    sem, sem_a, sem_b):
    wid = lax.axis_index("s") * NC + lax.axis_index("c")
    base = wid * BPW

    pltpu.sync_copy(center_h.at[pl.ds(base, BPW)], idxs.at[pl.ds(0, BPW)])
    pltpu.sync_copy(context_h.at[pl.ds(base, BPW)], idxs.at[pl.ds(BPW, BPW)])
    for k in range(K):
        pltpu.sync_copy(negT_h.at[pl.ds(k * B + base, BPW)],
                        idxs.at[pl.ds((2 + k) * BPW, BPW)])

    def tbody(j, carry):
        x = idxs[pl.ds(j * LANES, LANES)]
        p = lax.bitwise_and(x, TCHUNK - 1)
        hi = jnp.where(p >= H, 1, 0)
        idxs[pl.ds(j * LANES, LANES)] = (
            (x - p) + lax.shift_left(jnp.where(p >= H, p - H, p), 1) + hi)
        return carry

    lax.fori_loop(0, IDX_N // LANES, tbody, 0)

    def _gather_rows(slot, dst, dsem):
        return pltpu.async_copy(
            wx_h.at[idxs.at[pl.ds(slot * BPW, BPW)]], dst, dsem)

    pltpu.async_copy(wc_h.at[idxs.at[pl.ds(0, BPW)]], crows, sem).wait()
    _gather_rows(1, xrows_a, sem_a)
    _gather_rows(2, xrows_b, sem_b)

    pltpu.make_async_copy(wx_h.at[pl.ds(0, BPW)], xrows_a, sem_a).wait()
    _dot_rows(crows, xrows_a, sbuf, ssum)
    pltpu.sync_copy(sbuf, pos_out.at[pl.ds(base, BPW)])

    def kbody(j, carry):
        k0 = 2 * j
        k1 = 2 * j + 1
        k2 = jnp.minimum(2 * j + 2, K - 1)
        pltpu.make_async_copy(wx_h.at[pl.ds(0, BPW)], xrows_b, sem_b).wait()
        _gather_rows(2 + k1, xrows_a, sem_a)
        _dot_rows(crows, xrows_b, sbuf, ssum)
        pltpu.sync_copy(sbuf, neg_out.at[pl.ds(k0 * B + base, BPW)])
        pltpu.make_async_copy(wx_h.at[pl.ds(0, BPW)], xrows_a, sem_a).wait()
        _gather_rows(2 + k2, xrows_b, sem_b)
        _dot_rows(crows, xrows_a, sbuf, ssum)
        pltpu.sync_copy(sbuf, neg_out.at[pl.ds(k1 * B + base, BPW)])
        return carry

    lax.fori_loop(0, K // 2, kbody, 0)
    pltpu.make_async_copy(wx_h.at[pl.ds(0, BPW)], xrows_b, sem_b).wait()


_sc_scores = functools.partial(
    pl.kernel,
    out_type=[jax.ShapeDtypeStruct((B,), jnp.float32),
              jax.ShapeDtypeStruct((K * B,), jnp.float32)],
    mesh=plsc.VectorSubcoreMesh(core_axis_name="c", subcore_axis_name="s"),
    compiler_params=pltpu.CompilerParams(
        needs_layout_passes=False, use_tc_tiling_on_sc=False),
    scratch_types=[
        pltpu.VMEM((IDX_N,), jnp.int32),
        pltpu.VMEM((BPW, D), jnp.bfloat16),
        pltpu.VMEM((BPW, D), jnp.bfloat16),
        pltpu.VMEM((BPW, D), jnp.bfloat16),
        pltpu.VMEM((BPW,), jnp.float32),
        pltpu.VMEM((LANES * LANES,), jnp.float32),
        pltpu.SemaphoreType.DMA,
        pltpu.SemaphoreType.DMA,
        pltpu.SemaphoreType.DMA,
    ],
)(_sc_scores_body)


TCHUNK = 32768
H = TCHUNK // 2
NBLK = (1000000 + TCHUNK - 1) // TCHUNK
VPACK = NBLK * TCHUNK


def _transpose_body(wt_ref, out_ref):
    t = wt_ref[...].T.astype(jnp.bfloat16)
    out_ref[:, :D] = t[:H, :]
    out_ref[:, D:] = t[H:, :]


def _tc_relayout(w):
    wt = w.T
    packed = pl.pallas_call(
        _transpose_body,
        grid=(NBLK,),
        in_specs=[pl.BlockSpec((D, TCHUNK), lambda i: (0, i))],
        out_specs=pl.BlockSpec((H, 2 * D), lambda i: (i, 0)),
        out_shape=jax.ShapeDtypeStruct((NBLK * H, 2 * D), jnp.bfloat16),
    )(wt)
    return packed.reshape(VPACK, D)


def _loss_body(pos_ref, neg_ref, out_ref):
    pos = pos_ref[...]
    neg = neg_ref[...]
    s = jnp.sum(jax.nn.log_sigmoid(pos)) + jnp.sum(jax.nn.log_sigmoid(-neg))
    out_ref[0, 0] = -s / B


def _tc_loss(pos2d, neg2d):
    return pl.pallas_call(
        _loss_body,
        out_shape=jax.ShapeDtypeStruct((1, 1), jnp.float32),
        out_specs=pl.BlockSpec(memory_space=pltpu.SMEM),
    )(pos2d, neg2d)


def kernel(center, context, negatives, W_center, W_context):
    center = center.astype(jnp.int32)
    context = context.astype(jnp.int32)
    negT = negatives.astype(jnp.int32).T.reshape(K * B)
    pos, negs = _sc_scores(center, context, negT,
                           _tc_relayout(W_center), _tc_relayout(W_context))
    loss = _tc_loss(pos.reshape(B // 128, 128), negs.reshape(K * B // 128, 128))
    return loss[0, 0]


# speedup vs baseline: 1.9095x; 1.9095x over previous
import functools

import jax
import jax.numpy as jnp
from jax import lax
from jax.experimental import pallas as pl
from jax.experimental.pallas import tpu as pltpu
from jax.experimental.pallas import tpu_sc as plsc

B = 16384
K = 20
D = 64
LANES = 16
NC = 2
NS = 16
NW = NC * NS
BPW = B // NW
G = BPW // LANES

TCHUNK = 32768
WPR = D // 2
RQ = TCHUNK // 4
NBLK = (1000000 + TCHUNK - 1) // TCHUNK
VPACK = NBLK * TCHUNK


def _dot_rows(crows, xrows, sbuf, ssum):
    iota = lax.iota(jnp.int32, LANES)
    rots = [r * LANES + lax.bitwise_and(iota + r, LANES - 1)
            for r in range(LANES)]
    diags = [iota * LANES + lax.bitwise_and(iota + c, LANES - 1)
             for c in range(LANES)]

    def body(g, carry):
        rb = g * LANES
        for r in range(LANES):
            acc = None
            for q in range(WPR // LANES):
                cw = plsc.bitcast(crows[rb + r, pl.ds(q * LANES, LANES)],
                                  jnp.int32)
                xw = plsc.bitcast(xrows[rb + r, pl.ds(q * LANES, LANES)],
                                  jnp.int32)
                c_lo = plsc.bitcast(lax.shift_left(cw, 16), jnp.float32)
                x_lo = plsc.bitcast(lax.shift_left(xw, 16), jnp.float32)
                c_hi = plsc.bitcast(
                    lax.bitwise_and(cw, jnp.int32(-65536)), jnp.float32)
                x_hi = plsc.bitcast(
                    lax.bitwise_and(xw, jnp.int32(-65536)), jnp.float32)
                term = c_lo * x_lo + c_hi * x_hi
                acc = term if acc is None else acc + term
            plsc.store_scatter(ssum, [rots[r]], acc)
        tot = jnp.zeros((LANES,), jnp.float32)
        for c in range(LANES):
            tot = tot + plsc.load_gather(ssum, [diags[c]])
        sbuf[pl.ds(rb, LANES)] = tot
        return carry

    lax.fori_loop(0, G, body, 0)


IDX_N = (K + 2) * BPW


def _sc_scores_body(center_h, context_h, negT_h, wc_h, wx_h,
                    pos_out, neg_out,
                    idxs, crows, xrows_a, xrows_b, sbuf, ssum,
                    sem, sem_a, sem_b):
    wid = lax.axis_index("s") * NC + lax.axis_index("c")
    base = wid * BPW

    pltpu.sync_copy(center_h.at[pl.ds(base, BPW)], idxs.at[pl.ds(0, BPW)])
    pltpu.sync_copy(context_h.at[pl.ds(base, BPW)], idxs.at[pl.ds(BPW, BPW)])
    for k in range(K):
        pltpu.sync_copy(negT_h.at[pl.ds(k * B + base, BPW)],
                        idxs.at[pl.ds((2 + k) * BPW, BPW)])

    def tbody(j, carry):
        x = idxs[pl.ds(j * LANES, LANES)]
        p = lax.bitwise_and(x, TCHUNK - 1)
        idxs[pl.ds(j * LANES, LANES)] = (
            (x - p) + lax.shift_left(lax.bitwise_and(p, RQ - 1), 2)
            + lax.shift_right_logical(p, 13))
        return carry

    lax.fori_loop(0, IDX_N // LANES, tbody, 0)

    def _gather_rows(slot, dst, dsem):
        return pltpu.async_copy(
            wx_h.at[idxs.at[pl.ds(slot * BPW, BPW)]], dst, dsem)

    pltpu.async_copy(wc_h.at[idxs.at[pl.ds(0, BPW)]], crows, sem).wait()
    _gather_rows(1, xrows_a, sem_a)
    _gather_rows(2, xrows_b, sem_b)

    pltpu.make_async_copy(wx_h.at[pl.ds(0, BPW)], xrows_a, sem_a).wait()
    _dot_rows(crows, xrows_a, sbuf, ssum)
    pltpu.sync_copy(sbuf, pos_out.at[pl.ds(base, BPW)])

    def kbody(j, carry):
        k0 = 2 * j
        k1 = 2 * j + 1
        k2 = jnp.minimum(2 * j + 2, K - 1)
        pltpu.make_async_copy(wx_h.at[pl.ds(0, BPW)], xrows_b, sem_b).wait()
        _gather_rows(2 + k1, xrows_a, sem_a)
        _dot_rows(crows, xrows_b, sbuf, ssum)
        pltpu.sync_copy(sbuf, neg_out.at[pl.ds(k0 * B + base, BPW)])
        pltpu.make_async_copy(wx_h.at[pl.ds(0, BPW)], xrows_a, sem_a).wait()
        _gather_rows(2 + k2, xrows_b, sem_b)
        _dot_rows(crows, xrows_a, sbuf, ssum)
        pltpu.sync_copy(sbuf, neg_out.at[pl.ds(k1 * B + base, BPW)])
        return carry

    lax.fori_loop(0, K // 2, kbody, 0)
    pltpu.make_async_copy(wx_h.at[pl.ds(0, BPW)], xrows_b, sem_b).wait()


_sc_scores = functools.partial(
    pl.kernel,
    out_type=[jax.ShapeDtypeStruct((B,), jnp.float32),
              jax.ShapeDtypeStruct((K * B,), jnp.float32)],
    mesh=plsc.VectorSubcoreMesh(core_axis_name="c", subcore_axis_name="s"),
    compiler_params=pltpu.CompilerParams(
        needs_layout_passes=False, use_tc_tiling_on_sc=False),
    scratch_types=[
        pltpu.VMEM((IDX_N,), jnp.int32),
        pltpu.VMEM((BPW, WPR), jnp.float32),
        pltpu.VMEM((BPW, WPR), jnp.float32),
        pltpu.VMEM((BPW, WPR), jnp.float32),
        pltpu.VMEM((BPW,), jnp.float32),
        pltpu.VMEM((LANES * LANES,), jnp.float32),
        pltpu.SemaphoreType.DMA,
        pltpu.SemaphoreType.DMA,
        pltpu.SemaphoreType.DMA,
    ],
)(_sc_scores_body)


def _transpose_body(wt_ref, out_ref):
    t = wt_ref[...].T.astype(jnp.bfloat16)
    lo = lax.convert_element_type(
        lax.bitcast_convert_type(t[:, :D // 2], jnp.uint16), jnp.uint32)
    hi = lax.convert_element_type(
        lax.bitcast_convert_type(t[:, D // 2:], jnp.uint16), jnp.uint32)
    w = lax.bitcast_convert_type(
        lax.bitwise_or(lo, lax.shift_left(hi, jnp.uint32(16))), jnp.float32)
    for q in range(4):
        out_ref[:, q * WPR:(q + 1) * WPR] = w[q * RQ:(q + 1) * RQ, :]


def _tc_relayout(w):
    wt = w.T
    packed = pl.pallas_call(
        _transpose_body,
        grid=(NBLK,),
        in_specs=[pl.BlockSpec((D, TCHUNK), lambda i: (0, i))],
        out_specs=pl.BlockSpec((RQ, 4 * WPR), lambda i: (i, 0)),
        out_shape=jax.ShapeDtypeStruct((NBLK * RQ, 4 * WPR), jnp.float32),
    )(wt)
    return packed.reshape(VPACK, WPR)


def _loss_body(pos_ref, neg_ref, out_ref):
    pos = pos_ref[...]
    neg = neg_ref[...]
    s = jnp.sum(jax.nn.log_sigmoid(pos)) + jnp.sum(jax.nn.log_sigmoid(-neg))
    out_ref[0, 0] = -s / B


def _tc_loss(pos2d, neg2d):
    return pl.pallas_call(
        _loss_body,
        out_shape=jax.ShapeDtypeStruct((1, 1), jnp.float32),
        out_specs=pl.BlockSpec(memory_space=pltpu.SMEM),
    )(pos2d, neg2d)


def kernel(center, context, negatives, W_center, W_context):
    center = center.astype(jnp.int32)
    context = context.astype(jnp.int32)
    negT = negatives.astype(jnp.int32).T.reshape(K * B)
    pos, negs = _sc_scores(center, context, negT,
                           _tc_relayout(W_center), _tc_relayout(W_context))
    loss = _tc_loss(pos.reshape(B // 128, 128), negs.reshape(K * B // 128, 128))
    return loss[0, 0]


# speedup vs baseline: 2.2228x; 1.1641x over previous
import functools

import jax
import jax.numpy as jnp
from jax import lax
from jax.experimental import pallas as pl
from jax.experimental.pallas import tpu as pltpu
from jax.experimental.pallas import tpu_sc as plsc

B = 16384
K = 20
D = 64
LANES = 16
NC = 2
NS = 16
NW = NC * NS
BPW = B // NW
G = BPW // LANES


def _dot_rows(crows, xrows, sbuf, ssum):
    iota = lax.iota(jnp.int32, LANES)
    rots = [r * LANES + lax.bitwise_and(iota + r, LANES - 1)
            for r in range(LANES)]
    diags = [iota * LANES + lax.bitwise_and(iota + c, LANES - 1)
             for c in range(LANES)]

    def body(g, carry):
        rb = g * LANES
        for r in range(LANES):
            acc = (crows[rb + r, pl.ds(0, LANES)]
                   * xrows[rb + r, pl.ds(0, LANES)])
            for q in range(1, D // LANES):
                acc = acc + (crows[rb + r, pl.ds(q * LANES, LANES)]
                             * xrows[rb + r, pl.ds(q * LANES, LANES)])
            plsc.store_scatter(ssum, [rots[r]], acc)
        tot = jnp.zeros((LANES,), jnp.float32)
        for c in range(LANES):
            tot = tot + plsc.load_gather(ssum, [diags[c]])
        sbuf[pl.ds(rb, LANES)] = tot
        return carry

    lax.fori_loop(0, G, body, 0)


IDX_N = (K + 2) * BPW


def _sc_scores_body(center_h, context_h, negT_h, wc_h, wx_h,
                    pos_out, neg_out,
                    idxs, crows, xrows_a, xrows_b, sbuf, ssum,
                    sem, sem_a, sem_b):
    wid = lax.axis_index("s") * NC + lax.axis_index("c")
    base = wid * BPW

    pltpu.async_copy(center_h.at[pl.ds(base, BPW)],
                     idxs.at[pl.ds(0, BPW)], sem)
    pltpu.async_copy(context_h.at[pl.ds(base, BPW)],
                     idxs.at[pl.ds(BPW, BPW)], sem)
    for k in range(K):
        pltpu.async_copy(negT_h.at[pl.ds(k * B + base, BPW)],
                         idxs.at[pl.ds((2 + k) * BPW, BPW)], sem)
    pltpu.make_async_copy(negT_h.at[pl.ds(0, IDX_N)], idxs, sem).wait()

    def tbody(j, carry):
        x = idxs[pl.ds(j * LANES, LANES)]
        p = lax.bitwise_and(x, TCHUNK - 1)
        hi = jnp.where(p >= H, 1, 0)
        idxs[pl.ds(j * LANES, LANES)] = (
            (x - p) + lax.shift_left(jnp.where(p >= H, p - H, p), 1) + hi)
        return carry

    lax.fori_loop(0, IDX_N // LANES, tbody, 0)

    def _gather_rows(slot, dst, dsem):
        return pltpu.async_copy(
            wx_h.at[idxs.at[pl.ds(slot * BPW, BPW)]], dst, dsem)

    pltpu.async_copy(wc_h.at[idxs.at[pl.ds(0, BPW)]], crows, sem).wait()
    _gather_rows(1, xrows_a, sem_a)
    _gather_rows(2, xrows_b, sem_b)

    pltpu.make_async_copy(wx_h.at[pl.ds(0, BPW)], xrows_a, sem_a).wait()
    _dot_rows(crows, xrows_a, sbuf, ssum)
    pltpu.sync_copy(sbuf, pos_out.at[pl.ds(base, BPW)])

    def kbody(j, carry):
        k0 = 2 * j
        k1 = 2 * j + 1
        k2 = jnp.minimum(2 * j + 2, K - 1)
        pltpu.make_async_copy(wx_h.at[pl.ds(0, BPW)], xrows_b, sem_b).wait()
        _gather_rows(2 + k1, xrows_a, sem_a)
        _dot_rows(crows, xrows_b, sbuf, ssum)
        pltpu.sync_copy(sbuf, neg_out.at[pl.ds(k0 * B + base, BPW)])
        pltpu.make_async_copy(wx_h.at[pl.ds(0, BPW)], xrows_a, sem_a).wait()
        _gather_rows(2 + k2, xrows_b, sem_b)
        _dot_rows(crows, xrows_a, sbuf, ssum)
        pltpu.sync_copy(sbuf, neg_out.at[pl.ds(k1 * B + base, BPW)])
        return carry

    lax.fori_loop(0, K // 2, kbody, 0)
    pltpu.make_async_copy(wx_h.at[pl.ds(0, BPW)], xrows_b, sem_b).wait()


_sc_scores = functools.partial(
    pl.kernel,
    out_type=[jax.ShapeDtypeStruct((B,), jnp.float32),
              jax.ShapeDtypeStruct((K * B,), jnp.float32)],
    mesh=plsc.VectorSubcoreMesh(core_axis_name="c", subcore_axis_name="s"),
    compiler_params=pltpu.CompilerParams(
        needs_layout_passes=False, use_tc_tiling_on_sc=False),
    scratch_types=[
        pltpu.VMEM((IDX_N,), jnp.int32),
        pltpu.VMEM((BPW, D), jnp.float32),
        pltpu.VMEM((BPW, D), jnp.float32),
        pltpu.VMEM((BPW, D), jnp.float32),
        pltpu.VMEM((BPW,), jnp.float32),
        pltpu.VMEM((LANES * LANES,), jnp.float32),
        pltpu.SemaphoreType.DMA,
        pltpu.SemaphoreType.DMA,
        pltpu.SemaphoreType.DMA,
    ],
)(_sc_scores_body)


TCHUNK = 32768
H = TCHUNK // 2
NBLK = (1000000 + TCHUNK - 1) // TCHUNK
VPACK = NBLK * TCHUNK


def _transpose_body(wt_ref, out_ref):
    t = wt_ref[...].T
    out_ref[:, :D] = t[:H, :]
    out_ref[:, D:] = t[H:, :]


def _tc_relayout(w):
    wt = w.T
    packed = pl.pallas_call(
        _transpose_body,
        grid=(NBLK,),
        in_specs=[pl.BlockSpec((D, TCHUNK), lambda i: (0, i))],
        out_specs=pl.BlockSpec((H, 2 * D), lambda i: (i, 0)),
        out_shape=jax.ShapeDtypeStruct((NBLK * H, 2 * D), jnp.float32),
    )(wt)
    return packed.reshape(VPACK, D)


def _loss_body(pos_ref, neg_ref, out_ref):
    pos = pos_ref[...]
    neg = neg_ref[...]
    s = jnp.sum(jax.nn.log_sigmoid(pos)) + jnp.sum(jax.nn.log_sigmoid(-neg))
    out_ref[0, 0] = -s / B


def _tc_loss(pos2d, neg2d):
    return pl.pallas_call(
        _loss_body,
        out_shape=jax.ShapeDtypeStruct((1, 1), jnp.float32),
        out_specs=pl.BlockSpec(memory_space=pltpu.SMEM),
    )(pos2d, neg2d)


def kernel(center, context, negatives, W_center, W_context):
    center = center.astype(jnp.int32)
    context = context.astype(jnp.int32)
    negT = negatives.astype(jnp.int32).T.reshape(K * B)
    pos, negs = _sc_scores(center, context, negT,
                           _tc_relayout(W_center), _tc_relayout(W_context))
    loss = _tc_loss(pos.reshape(B // 128, 128), negs.reshape(K * B // 128, 128))
    return loss[0, 0]
